# baseline (speedup 1.0000x reference)
import functools

import jax
import jax.numpy as jnp
from jax import lax
from jax.experimental import pallas as pl
from jax.experimental.pallas import tpu as pltpu
from jax.experimental.pallas import tpu_sc as plsc

N = 50000
E = 800000
D = 58
DP = 64
HID = 128
LAT = 32
G = 1024

NC = 2
NS = 16
EP = 819200
EPT = EP // NS
NACC = 51200
RPT = NACC // NS
GACC = 1040
GPT = GACC // NS
NPAD = 51200
NPT = NPAD // NS

LS = 256
NB2 = 4
ECH = 2048
DCH = 2048
PCH = 320

_mesh = plsc.VectorSubcoreMesh(
    core_axis_name="c", subcore_axis_name="s", num_cores=NC, num_subcores=NS)
_sc_params = pltpu.CompilerParams(use_tc_tiling_on_sc=False)


def _zero_vmem(ref, n_f32):
    z = jnp.zeros((16,), jnp.float32)
    if ref.ndim == 1:
        def zb(i, _):
            ref[pl.ds(pl.multiple_of(i * 16, 16), 16)] = z
            return _
        lax.fori_loop(0, n_f32 // 16, zb, None)
    else:
        rows, cols = ref.shape

        def zb(r, _):
            for j in range(cols // 16):
                ref[r, pl.ds(j * 16, 16)] = z
            return _
        lax.fori_loop(0, rows, zb, None)


def _deg_body(dst1, deg_out, acc, idx, ones_v, zb, sem):
    c = lax.axis_index("c")
    s = lax.axis_index("s")
    base = pl.multiple_of(s * RPT, 8)
    _zero_vmem(zb, RPT)
    one = jnp.ones((16,), jnp.float32)

    def of(i, _):
        ones_v[pl.ds(pl.multiple_of(i * 16, 16), 16)] = one
        return _

    lax.fori_loop(0, DCH // 16, of, None)
    pltpu.sync_copy(zb, acc.at[pl.ds(base, RPT)])
    plsc.subcore_barrier()
    pltpu.sync_copy(dst1.at[c, pl.ds(s * EPT, EPT)], idx)
    def db(b, _):
        pltpu.async_copy(
            ones_v,
            acc.at[idx.at[pl.ds(pl.multiple_of(b * DCH, 8), DCH)]],
            sem, add=True).wait()
        return _

    lax.fori_loop(0, EPT // DCH, db, None)
    plsc.subcore_barrier()
    pltpu.sync_copy(acc.at[pl.ds(base, RPT)], zb)
    pltpu.sync_copy(zb, deg_out.at[c, pl.ds(base, RPT)])


_deg_kernel = functools.partial(
    pl.kernel,
    out_type=jax.ShapeDtypeStruct((NC, NACC), jnp.float32),
    mesh=_mesh,
    compiler_params=_sc_params,
    scratch_types=[
        pltpu.VMEM_SHARED((NACC,), jnp.float32),
        pltpu.VMEM((EPT,), jnp.int32),
        pltpu.VMEM((DCH,), jnp.float32),
        pltpu.VMEM((RPT,), jnp.float32),
        pltpu.SemaphoreType.DMA,
    ],
)(_deg_body)


def _edge_pass(tab, src1, dst1, out, acc, idx_s, idx_d, rbuf, zrow,
               semg, sems, s):
    w = zrow.shape[1]
    base = pl.multiple_of(s * RPT, 8)
    _zero_vmem(zrow, 128 * w)

    def zacc(i, _):
        pltpu.sync_copy(
            zrow, acc.at[pl.ds(pl.multiple_of(base + i * 128, 8), 128)])
        return _

    lax.fori_loop(0, RPT // 128, zacc, None)
    plsc.subcore_barrier()

    def step_fn(step, _):
        e0 = pl.multiple_of(s * EPT + step * ECH, 8)
        pltpu.sync_copy(src1.at[pl.ds(e0, ECH)], idx_s)
        pltpu.sync_copy(dst1.at[pl.ds(e0, ECH)], idx_d)

        def batch(b, _):
            r = pl.multiple_of(b * (NB2 * LS), LS)
            gd = [
                pltpu.async_copy(tab.at[idx_s.at[pl.ds(r + j * LS, LS)]],
                                 rbuf.at[j], semg)
                for j in range(NB2)
            ]
            for d in gd:
                d.wait()
            sd = [
                pltpu.async_copy(rbuf.at[j],
                                 acc.at[idx_d.at[pl.ds(r + j * LS, LS)]],
                                 sems, add=True)
                for j in range(NB2)
            ]
            for d in sd:
                d.wait()
            return _

        lax.fori_loop(0, ECH // (NB2 * LS), batch, None)
        return _

    lax.fori_loop(0, EPT // ECH, step_fn, None)
    plsc.subcore_barrier()

    def wb(i, _):
        o = pl.multiple_of(base + i * 128, 8)
        pltpu.sync_copy(acc.at[pl.ds(o, 128)], zrow)
        pltpu.sync_copy(zrow, out.at[pl.ds(o, 128)])
        return _

    lax.fori_loop(0, RPT // 128, wb, None)


_conv_scratch = [
    pltpu.VMEM_SHARED((N, 16), jnp.float32),
    pltpu.VMEM_SHARED((NACC, 16), jnp.float32),
    pltpu.VMEM((125, 16), jnp.float32),
    pltpu.VMEM((ECH,), jnp.int32),
    pltpu.VMEM((ECH,), jnp.int32),
    pltpu.VMEM((NB2, LS, 16), jnp.float32),
    pltpu.VMEM((128, 16), jnp.float32),
    pltpu.SemaphoreType.DMA,
    pltpu.SemaphoreType.DMA,
]


def _spmem_pass(tab_hbm, src1, dst1, out, tabsp, acc, sbuf, idx_s, idx_d,
                rbuf, zrow, semg, sems, s):
    nrs = N // NS

    def stage(i, _):
        r0 = s * nrs + i * 125
        pltpu.sync_copy(tab_hbm.at[pl.ds(r0, 125)], sbuf)
        pltpu.sync_copy(sbuf, tabsp.at[pl.ds(r0, 125)])
        return _

    lax.fori_loop(0, nrs // 125, stage, None)
    _edge_pass(tabsp, src1, dst1, out, acc, idx_s, idx_d, rbuf, zrow,
               semg, sems, s)


def _conv1_body(tabs, src1, dst1, s1, tabsp, acc, sbuf, idx_s, idx_d, rbuf,
                zrow, semg, sems):
    c = lax.axis_index("c")
    s = lax.axis_index("s")
    for g in range(2):
        for j in range(2):
            q = 2 * c + j
            _spmem_pass(tabs.at[g, q], src1.at[g], dst1.at[g], s1.at[g, q],
                        tabsp, acc, sbuf, idx_s, idx_d, rbuf, zrow,
                        semg, sems, s)
            plsc.subcore_barrier()


_conv1_kernel = functools.partial(
    pl.kernel,
    out_type=jax.ShapeDtypeStruct((2, 4, NACC, 16), jnp.float32),
    mesh=_mesh,
    compiler_params=_sc_params,
    scratch_types=_conv_scratch,
)(_conv1_body)


def _conv2_body(tabs, src1, dst1, s2, tabsp, acc, sbuf, idx_s, idx_d, rbuf,
                zrow, semg, sems):
    c = lax.axis_index("c")
    s = lax.axis_index("s")
    for p in range(2):
        _spmem_pass(tabs.at[c, p], src1.at[c], dst1.at[c], s2.at[c, p],
                    tabsp, acc, sbuf, idx_s, idx_d, rbuf, zrow,
                    semg, sems, s)
        if p == 0:
            plsc.subcore_barrier()


_conv2_kernel = functools.partial(
    pl.kernel,
    out_type=jax.ShapeDtypeStruct((NC, 2, NACC, 16), jnp.float32),
    mesh=_mesh,
    compiler_params=_sc_params,
    scratch_types=_conv_scratch,
)(_conv2_body)


def _pool_body(vext, bat1, pool_out, acc, idx, vbuf, zrow, semg, sems):
    c = lax.axis_index("c")
    s = lax.axis_index("s")
    _zero_vmem(zrow, GPT * 48)
    pltpu.sync_copy(zrow, acc.at[pl.ds(s * GPT, GPT)])
    plsc.subcore_barrier()
    pltpu.sync_copy(bat1.at[c, pl.ds(s * NPT, NPT)], idx)
    def pb(b, _):
        for j in range(2):
            bb = b * 2 + j
            o = pl.multiple_of(bb * PCH, 8)
            pltpu.async_copy(vext.at[c, pl.ds(s * NPT + o, PCH)],
                             vbuf.at[j], semg).wait()
            pltpu.async_copy(vbuf.at[j], acc.at[idx.at[pl.ds(o, PCH)]],
                             sems, add=True).wait()
        return _

    lax.fori_loop(0, NPT // (2 * PCH), pb, None)
    plsc.subcore_barrier()
    pltpu.sync_copy(acc.at[pl.ds(s * GPT, GPT)], zrow)
    pltpu.sync_copy(zrow, pool_out.at[c, pl.ds(s * GPT, GPT)])


_pool_kernel = functools.partial(
    pl.kernel,
    out_type=jax.ShapeDtypeStruct((NC, GACC, 48), jnp.float32),
    mesh=_mesh,
    compiler_params=_sc_params,
    scratch_types=[
        pltpu.VMEM_SHARED((GACC, 48), jnp.float32),
        pltpu.VMEM((NPT,), jnp.int32),
        pltpu.VMEM((2, PCH, 48), jnp.float32),
        pltpu.VMEM((GPT, 48), jnp.float32),
        pltpu.SemaphoreType.DMA,
        pltpu.SemaphoreType.DMA,
    ],
)(_pool_body)


def _mid_body(y_ref, w1_ref, b1_ref, wmu_ref, u_ref):
    y = y_ref[0]
    h = jnp.maximum(
        jnp.dot(y, w1_ref[...], preferred_element_type=jnp.float32)
        + b1_ref[...], 0.0)
    u_ref[0] = jnp.dot(h, wmu_ref[...], preferred_element_type=jnp.float32)


def _mid_call(y1pre, w1p, b1, wmu):
    blk = 2000
    return pl.pallas_call(
        _mid_body,
        grid=(2, N // blk),
        in_specs=[
            pl.BlockSpec((1, blk, DP), lambda g, i: (g, i, 0)),
            pl.BlockSpec((DP, HID), lambda g, i: (0, 0)),
            pl.BlockSpec((1, HID), lambda g, i: (0, 0)),
            pl.BlockSpec((HID, LAT), lambda g, i: (0, 0)),
        ],
        out_specs=pl.BlockSpec((1, blk, LAT), lambda g, i: (g, i, 0)),
        out_shape=jax.ShapeDtypeStruct((2, N, LAT), jnp.float32),
    )(y1pre, w1p, b1.reshape(1, HID), wmu)


def _head_body(h_ref, r1_ref, rb1_ref, r2_ref, rb2_ref, r3_ref, rb3_ref,
               o_ref):
    h = jnp.maximum(
        jnp.dot(h_ref[...], r1_ref[...], preferred_element_type=jnp.float32)
        + rb1_ref[...], 0.0)
    h = jnp.maximum(
        jnp.dot(h, r2_ref[...], preferred_element_type=jnp.float32)
        + rb2_ref[...], 0.0)
    o_ref[...] = (
        jnp.dot(h, r3_ref[...], preferred_element_type=jnp.float32)
        + rb3_ref[...])


def _head_call(hcat, R1, rb1, R2, rb2, R3, rb3):
    return pl.pallas_call(
        _head_body,
        out_shape=jax.ShapeDtypeStruct((G, 2), jnp.float32),
    )(hcat, R1, rb1.reshape(1, -1), R2, rb2.reshape(1, -1), R3,
      rb3.reshape(1, -1))


def _pad_to(a, n, lo, hi):
    pad = lo + jnp.arange(n - a.shape[0], dtype=a.dtype) % (hi - lo)
    return jnp.concatenate([a, pad])


def kernel(xA, edge_indexA, batchA, xB, edge_indexB, batchB, W1, b1, Wmu, bmu,
           Wstd, bstd, R1, rb1, R2, rb2, R3, rb3):
    src1 = jnp.stack([_pad_to(edge_indexA[0], EP, 0, N),
                      _pad_to(edge_indexB[0], EP, 0, N)])
    dst1 = jnp.stack([_pad_to(edge_indexA[1], EP, N, NACC),
                      _pad_to(edge_indexB[1], EP, N, NACC)])
    bat1 = jnp.stack([_pad_to(batchA, NPAD, G, GACC),
                      _pad_to(batchB, NPAD, G, GACC)])

    degs = _deg_kernel(dst1)[:, :N]
    dinv = lax.rsqrt(degs + 1.0)

    x = jnp.stack([xA, xB])
    xp = jnp.pad(x * dinv[:, :, None], ((0, 0), (0, 0), (0, DP - D)))
    tabs1 = jnp.stack([xp[:, :, 16 * q:16 * (q + 1)] for q in range(4)],
                      axis=1)
    s1 = _conv1_kernel(tabs1, src1, dst1)
    s1f = jnp.concatenate([s1[:, q, :N] for q in range(4)], axis=-1)
    y1pre = dinv[:, :, None] * (s1f + xp)

    w1p = jnp.pad(W1, ((0, DP - D), (0, 0)))
    u = _mid_call(y1pre, w1p, b1, Wmu)
    up = dinv[:, :, None] * u

    uph = jnp.stack([up[:, :, :16], up[:, :, 16:]], axis=1)
    s2h = _conv2_kernel(uph, src1, dst1)
    s2 = jnp.concatenate([s2h[:, 0, :N], s2h[:, 1, :N]], axis=-1)
    v = dinv[:, :, None] * (s2 + up)
    vext = jnp.concatenate([
        jnp.pad(v, ((0, 0), (0, NPAD - N), (0, 0))),
        jnp.pad(jnp.ones((2, N, 1), jnp.float32),
                ((0, 0), (0, NPAD - N), (0, 0))),
        jnp.zeros((2, NPAD, 15), jnp.float32),
    ], axis=-1)
    pooled = _pool_kernel(vext, bat1)
    sums = pooled[:, :G, :LAT]
    cnt = pooled[:, :G, LAT]
    z = (sums + cnt[:, :, None] * bmu) / jnp.maximum(cnt, 1.0)[:, :, None]

    hcat = jnp.concatenate([z[0], z[1]], axis=1)
    return _head_call(hcat, R1, rb1, R2, rb2, R3, rb3)


# speedup vs baseline: 1.2382x; 1.2382x over previous
import functools

import jax
import jax.numpy as jnp
from jax import lax
from jax.experimental import pallas as pl
from jax.experimental.pallas import tpu as pltpu
from jax.experimental.pallas import tpu_sc as plsc

N = 50000
E = 800000
D = 58
DP = 64
HID = 128
LAT = 32
G = 1024

NC = 2
NS = 16
EP = 819200
EPT = EP // NS
NACC = 51200
RPT = NACC // NS
GACC = 1040
GPT = GACC // NS
NPAD = 51200
NPT = NPAD // NS

LS = 256
NB2 = 4
ECH = 2048
DCH = 2048
PCH = 320

_mesh = plsc.VectorSubcoreMesh(
    core_axis_name="c", subcore_axis_name="s", num_cores=NC, num_subcores=NS)
_sc_params = pltpu.CompilerParams(use_tc_tiling_on_sc=False)


def _zero_vmem(ref, n_f32):
    z = jnp.zeros((16,), jnp.float32)
    if ref.ndim == 1:
        def zb(i, _):
            ref[pl.ds(pl.multiple_of(i * 16, 16), 16)] = z
            return _
        lax.fori_loop(0, n_f32 // 16, zb, None)
    else:
        rows, cols = ref.shape

        def zb(r, _):
            for j in range(cols // 16):
                ref[r, pl.ds(j * 16, 16)] = z
            return _
        lax.fori_loop(0, rows, zb, None)


def _deg_body(dst1, deg_out, acc, idx, ones_v, zb, sem):
    c = lax.axis_index("c")
    s = lax.axis_index("s")
    base = pl.multiple_of(s * RPT, 8)
    _zero_vmem(zb, RPT)
    one = jnp.ones((16,), jnp.float32)

    def of(i, _):
        ones_v[pl.ds(pl.multiple_of(i * 16, 16), 16)] = one
        return _

    lax.fori_loop(0, DCH // 16, of, None)
    pltpu.sync_copy(zb, acc.at[pl.ds(base, RPT)])
    plsc.subcore_barrier()
    pltpu.sync_copy(dst1.at[c, pl.ds(s * EPT, EPT)], idx)
    def db(b, _):
        pltpu.async_copy(
            ones_v,
            acc.at[idx.at[pl.ds(pl.multiple_of(b * DCH, 8), DCH)]],
            sem, add=True).wait()
        return _

    lax.fori_loop(0, EPT // DCH, db, None)
    plsc.subcore_barrier()
    pltpu.sync_copy(acc.at[pl.ds(base, RPT)], zb)
    pltpu.sync_copy(zb, deg_out.at[c, pl.ds(base, RPT)])


_deg_kernel = functools.partial(
    pl.kernel,
    out_type=jax.ShapeDtypeStruct((NC, NACC), jnp.float32),
    mesh=_mesh,
    compiler_params=_sc_params,
    scratch_types=[
        pltpu.VMEM_SHARED((NACC,), jnp.float32),
        pltpu.VMEM((EPT,), jnp.int32),
        pltpu.VMEM((DCH,), jnp.float32),
        pltpu.VMEM((RPT,), jnp.float32),
        pltpu.SemaphoreType.DMA,
    ],
)(_deg_body)


def _edge_pass(tab, src1, dst1, out, acc, idx_s, idx_d, rbuf, zrow,
               semg, sems, s):
    w = zrow.shape[1]
    nb = rbuf.shape[0]
    base = pl.multiple_of(s * RPT, 8)
    _zero_vmem(zrow, 128 * w)

    def zacc(i, _):
        pltpu.sync_copy(
            zrow, acc.at[pl.ds(pl.multiple_of(base + i * 128, 8), 128)])
        return _

    lax.fori_loop(0, RPT // 128, zacc, None)
    plsc.subcore_barrier()

    def step_fn(step, _):
        e0 = pl.multiple_of(s * EPT + step * ECH, 8)
        pltpu.sync_copy(src1.at[pl.ds(e0, ECH)], idx_s)
        pltpu.sync_copy(dst1.at[pl.ds(e0, ECH)], idx_d)

        def batch(b, _):
            r = pl.multiple_of(b * (nb * LS), LS)
            gd = [
                pltpu.async_copy(tab.at[idx_s.at[pl.ds(r + j * LS, LS)]],
                                 rbuf.at[j], semg)
                for j in range(nb)
            ]
            for d in gd:
                d.wait()
            sd = [
                pltpu.async_copy(rbuf.at[j],
                                 acc.at[idx_d.at[pl.ds(r + j * LS, LS)]],
                                 sems, add=True)
                for j in range(nb)
            ]
            for d in sd:
                d.wait()
            return _

        lax.fori_loop(0, ECH // (nb * LS), batch, None)
        return _

    lax.fori_loop(0, EPT // ECH, step_fn, None)
    plsc.subcore_barrier()

    def wb(i, _):
        o = pl.multiple_of(base + i * 128, 8)
        pltpu.sync_copy(acc.at[pl.ds(o, 128)], zrow)
        pltpu.sync_copy(zrow, out.at[pl.ds(o, 128)])
        return _

    lax.fori_loop(0, RPT // 128, wb, None)


_conv_scratch = [
    pltpu.VMEM_SHARED((N, 16), jnp.float32),
    pltpu.VMEM_SHARED((NACC, 16), jnp.float32),
    pltpu.VMEM((125, 16), jnp.float32),
    pltpu.VMEM((ECH,), jnp.int32),
    pltpu.VMEM((ECH,), jnp.int32),
    pltpu.VMEM((NB2, LS, 16), jnp.float32),
    pltpu.VMEM((128, 16), jnp.float32),
    pltpu.SemaphoreType.DMA,
    pltpu.SemaphoreType.DMA,
]


def _spmem_pass(tab_hbm, src1, dst1, out, tabsp, acc, sbuf, idx_s, idx_d,
                rbuf, zrow, semg, sems, s):
    nrs = N // NS

    def stage(i, _):
        r0 = s * nrs + i * 125
        pltpu.sync_copy(tab_hbm.at[pl.ds(r0, 125)], sbuf)
        pltpu.sync_copy(sbuf, tabsp.at[pl.ds(r0, 125)])
        return _

    lax.fori_loop(0, nrs // 125, stage, None)
    _edge_pass(tabsp, src1, dst1, out, acc, idx_s, idx_d, rbuf, zrow,
               semg, sems, s)


def _conv1_body(tabs, src1, dst1, s1, acc, idx_s, idx_d, rbuf, zrow,
                semg, sems):
    c = lax.axis_index("c")
    s = lax.axis_index("s")
    for g in range(2):
        _edge_pass(tabs.at[g, c], src1.at[g], dst1.at[g], s1.at[g, c],
                   acc, idx_s, idx_d, rbuf, zrow, semg, sems, s)
        if g == 0:
            plsc.subcore_barrier()


_conv1_kernel = functools.partial(
    pl.kernel,
    out_type=jax.ShapeDtypeStruct((2, NC, NACC, 32), jnp.float32),
    mesh=_mesh,
    compiler_params=_sc_params,
    scratch_types=[
        pltpu.VMEM_SHARED((NACC, 32), jnp.float32),
        pltpu.VMEM((ECH,), jnp.int32),
        pltpu.VMEM((ECH,), jnp.int32),
        pltpu.VMEM((2, LS, 32), jnp.float32),
        pltpu.VMEM((128, 32), jnp.float32),
        pltpu.SemaphoreType.DMA,
        pltpu.SemaphoreType.DMA,
    ],
)(_conv1_body)


def _conv2_body(tabs, src1, dst1, s2, tabsp, acc, sbuf, idx_s, idx_d, rbuf,
                zrow, semg, sems):
    c = lax.axis_index("c")
    s = lax.axis_index("s")
    for p in range(2):
        _spmem_pass(tabs.at[c, p], src1.at[c], dst1.at[c], s2.at[c, p],
                    tabsp, acc, sbuf, idx_s, idx_d, rbuf, zrow,
                    semg, sems, s)
        if p == 0:
            plsc.subcore_barrier()


_conv2_kernel = functools.partial(
    pl.kernel,
    out_type=jax.ShapeDtypeStruct((NC, 2, NACC, 16), jnp.float32),
    mesh=_mesh,
    compiler_params=_sc_params,
    scratch_types=_conv_scratch,
)(_conv2_body)


def _pool_body(vext, bat1, pool_out, acc, idx, vbuf, zrow, semg, sems):
    c = lax.axis_index("c")
    s = lax.axis_index("s")
    _zero_vmem(zrow, GPT * 48)
    pltpu.sync_copy(zrow, acc.at[pl.ds(s * GPT, GPT)])
    plsc.subcore_barrier()
    pltpu.sync_copy(bat1.at[c, pl.ds(s * NPT, NPT)], idx)
    def pb(b, _):
        for j in range(2):
            bb = b * 2 + j
            o = pl.multiple_of(bb * PCH, 8)
            pltpu.async_copy(vext.at[c, pl.ds(s * NPT + o, PCH)],
                             vbuf.at[j], semg).wait()
            pltpu.async_copy(vbuf.at[j], acc.at[idx.at[pl.ds(o, PCH)]],
                             sems, add=True).wait()
        return _

    lax.fori_loop(0, NPT // (2 * PCH), pb, None)
    plsc.subcore_barrier()
    pltpu.sync_copy(acc.at[pl.ds(s * GPT, GPT)], zrow)
    pltpu.sync_copy(zrow, pool_out.at[c, pl.ds(s * GPT, GPT)])


_pool_kernel = functools.partial(
    pl.kernel,
    out_type=jax.ShapeDtypeStruct((NC, GACC, 48), jnp.float32),
    mesh=_mesh,
    compiler_params=_sc_params,
    scratch_types=[
        pltpu.VMEM_SHARED((GACC, 48), jnp.float32),
        pltpu.VMEM((NPT,), jnp.int32),
        pltpu.VMEM((2, PCH, 48), jnp.float32),
        pltpu.VMEM((GPT, 48), jnp.float32),
        pltpu.SemaphoreType.DMA,
        pltpu.SemaphoreType.DMA,
    ],
)(_pool_body)


def _mid_body(y_ref, w1_ref, b1_ref, wmu_ref, u_ref):
    y = y_ref[0]
    h = jnp.maximum(
        jnp.dot(y, w1_ref[...], preferred_element_type=jnp.float32)
        + b1_ref[...], 0.0)
    u_ref[0] = jnp.dot(h, wmu_ref[...], preferred_element_type=jnp.float32)


def _mid_call(y1pre, w1p, b1, wmu):
    blk = 2000
    return pl.pallas_call(
        _mid_body,
        grid=(2, N // blk),
        in_specs=[
            pl.BlockSpec((1, blk, DP), lambda g, i: (g, i, 0)),
            pl.BlockSpec((DP, HID), lambda g, i: (0, 0)),
            pl.BlockSpec((1, HID), lambda g, i: (0, 0)),
            pl.BlockSpec((HID, LAT), lambda g, i: (0, 0)),
        ],
        out_specs=pl.BlockSpec((1, blk, LAT), lambda g, i: (g, i, 0)),
        out_shape=jax.ShapeDtypeStruct((2, N, LAT), jnp.float32),
    )(y1pre, w1p, b1.reshape(1, HID), wmu)


def _head_body(h_ref, r1_ref, rb1_ref, r2_ref, rb2_ref, r3_ref, rb3_ref,
               o_ref):
    h = jnp.maximum(
        jnp.dot(h_ref[...], r1_ref[...], preferred_element_type=jnp.float32)
        + rb1_ref[...], 0.0)
    h = jnp.maximum(
        jnp.dot(h, r2_ref[...], preferred_element_type=jnp.float32)
        + rb2_ref[...], 0.0)
    o_ref[...] = (
        jnp.dot(h, r3_ref[...], preferred_element_type=jnp.float32)
        + rb3_ref[...])


def _head_call(hcat, R1, rb1, R2, rb2, R3, rb3):
    return pl.pallas_call(
        _head_body,
        out_shape=jax.ShapeDtypeStruct((G, 2), jnp.float32),
    )(hcat, R1, rb1.reshape(1, -1), R2, rb2.reshape(1, -1), R3,
      rb3.reshape(1, -1))


def _pad_to(a, n, lo, hi):
    pad = lo + jnp.arange(n - a.shape[0], dtype=a.dtype) % (hi - lo)
    return jnp.concatenate([a, pad])


def kernel(xA, edge_indexA, batchA, xB, edge_indexB, batchB, W1, b1, Wmu, bmu,
           Wstd, bstd, R1, rb1, R2, rb2, R3, rb3):
    src1 = jnp.stack([_pad_to(edge_indexA[0], EP, 0, N),
                      _pad_to(edge_indexB[0], EP, 0, N)])
    dst1 = jnp.stack([_pad_to(edge_indexA[1], EP, N, NACC),
                      _pad_to(edge_indexB[1], EP, N, NACC)])
    bat1 = jnp.stack([_pad_to(batchA, NPAD, G, GACC),
                      _pad_to(batchB, NPAD, G, GACC)])

    degs = _deg_kernel(dst1)[:, :N]
    dinv = lax.rsqrt(degs + 1.0)

    x = jnp.stack([xA, xB])
    xp = jnp.pad(x * dinv[:, :, None], ((0, 0), (0, 0), (0, DP - D)))
    tabs1 = jnp.stack([xp[:, :, :32], xp[:, :, 32:]], axis=1)
    s1 = _conv1_kernel(tabs1, src1, dst1)
    s1f = jnp.concatenate([s1[:, 0, :N], s1[:, 1, :N]], axis=-1)
    y1pre = dinv[:, :, None] * (s1f + xp)

    w1p = jnp.pad(W1, ((0, DP - D), (0, 0)))
    u = _mid_call(y1pre, w1p, b1, Wmu)
    up = dinv[:, :, None] * u

    uph = jnp.stack([up[:, :, :16], up[:, :, 16:]], axis=1)
    s2h = _conv2_kernel(uph, src1, dst1)
    s2 = jnp.concatenate([s2h[:, 0, :N], s2h[:, 1, :N]], axis=-1)
    v = dinv[:, :, None] * (s2 + up)
    vext = jnp.concatenate([
        jnp.pad(v, ((0, 0), (0, NPAD - N), (0, 0))),
        jnp.pad(jnp.ones((2, N, 1), jnp.float32),
                ((0, 0), (0, NPAD - N), (0, 0))),
        jnp.zeros((2, NPAD, 15), jnp.float32),
    ], axis=-1)
    pooled = _pool_kernel(vext, bat1)
    sums = pooled[:, :G, :LAT]
    cnt = pooled[:, :G, LAT]
    z = (sums + cnt[:, :, None] * bmu) / jnp.maximum(cnt, 1.0)[:, :, None]

    hcat = jnp.concatenate([z[0], z[1]], axis=1)
    return _head_call(hcat, R1, rb1, R2, rb2, R3, rb3)


# speedup vs baseline: 1.2690x; 1.0249x over previous
import functools

import jax
import jax.numpy as jnp
from jax import lax
from jax.experimental import pallas as pl
from jax.experimental.pallas import tpu as pltpu
from jax.experimental.pallas import tpu_sc as plsc

N = 50000
E = 800000
D = 58
DP = 64
HID = 128
LAT = 32
G = 1024

NC = 2
NS = 16
EP = 819200
EPT = EP // NS
NACC = 51200
RPT = NACC // NS
GACC = 1152
GPT = GACC // NS
NPAD = 51200
NPT = NPAD // NS

LS = 256
NB2 = 4
ECH = 2048
DCH = 2048
PCH = 320

_mesh = plsc.VectorSubcoreMesh(
    core_axis_name="c", subcore_axis_name="s", num_cores=NC, num_subcores=NS)
_sc_params = pltpu.CompilerParams(use_tc_tiling_on_sc=False)


def _zero_vmem(ref, n_f32):
    z = jnp.zeros((16,), jnp.float32)
    if ref.ndim == 1:
        def zb(i, _):
            ref[pl.ds(pl.multiple_of(i * 16, 16), 16)] = z
            return _
        lax.fori_loop(0, n_f32 // 16, zb, None)
    else:
        rows, cols = ref.shape

        def zb(r, _):
            for j in range(cols // 16):
                ref[r, pl.ds(j * 16, 16)] = z
            return _
        lax.fori_loop(0, rows, zb, None)


def _deg_body(dst1, deg_out, acc, idx, ones_v, zb, sem):
    c = lax.axis_index("c")
    s = lax.axis_index("s")
    base = pl.multiple_of(s * RPT, 8)
    _zero_vmem(zb, RPT)
    one = jnp.ones((16,), jnp.float32)

    def of(i, _):
        ones_v[pl.ds(pl.multiple_of(i * 16, 16), 16)] = one
        return _

    lax.fori_loop(0, DCH // 16, of, None)
    pltpu.sync_copy(zb, acc.at[pl.ds(base, RPT)])
    plsc.subcore_barrier()
    pltpu.sync_copy(dst1.at[c, pl.ds(s * EPT, EPT)], idx)
    def db(b, _):
        pltpu.async_copy(
            ones_v,
            acc.at[idx.at[pl.ds(pl.multiple_of(b * DCH, 8), DCH)]],
            sem, add=True).wait()
        return _

    lax.fori_loop(0, EPT // DCH, db, None)
    plsc.subcore_barrier()
    pltpu.sync_copy(acc.at[pl.ds(base, RPT)], zb)
    pltpu.sync_copy(zb, deg_out.at[c, pl.ds(base, RPT)])


_deg_kernel = functools.partial(
    pl.kernel,
    out_type=jax.ShapeDtypeStruct((NC, NACC), jnp.float32),
    mesh=_mesh,
    compiler_params=_sc_params,
    scratch_types=[
        pltpu.VMEM_SHARED((NACC,), jnp.float32),
        pltpu.VMEM((EPT,), jnp.int32),
        pltpu.VMEM((DCH,), jnp.float32),
        pltpu.VMEM((RPT,), jnp.float32),
        pltpu.SemaphoreType.DMA,
    ],
)(_deg_body)


def _edge_pass(tab, src1, dst1, out, acc, idx_s, idx_d, rbuf, zrow,
               semg, sems, s):
    w = zrow.shape[1]
    nb = rbuf.shape[0]
    base = pl.multiple_of(s * RPT, 8)
    _zero_vmem(zrow, 128 * w)

    def zacc(i, _):
        pltpu.sync_copy(
            zrow, acc.at[pl.ds(pl.multiple_of(base + i * 128, 8), 128)])
        return _

    lax.fori_loop(0, RPT // 128, zacc, None)
    plsc.subcore_barrier()

    def step_fn(step, _):
        e0 = pl.multiple_of(s * EPT + step * ECH, 8)
        pltpu.sync_copy(src1.at[pl.ds(e0, ECH)], idx_s)
        pltpu.sync_copy(dst1.at[pl.ds(e0, ECH)], idx_d)

        def batch(b, _):
            r = pl.multiple_of(b * (nb * LS), LS)
            gd = [
                pltpu.async_copy(tab.at[idx_s.at[pl.ds(r + j * LS, LS)]],
                                 rbuf.at[j], semg)
                for j in range(nb)
            ]
            for d in gd:
                d.wait()
            sd = [
                pltpu.async_copy(rbuf.at[j],
                                 acc.at[idx_d.at[pl.ds(r + j * LS, LS)]],
                                 sems, add=True)
                for j in range(nb)
            ]
            for d in sd:
                d.wait()
            return _

        lax.fori_loop(0, ECH // (nb * LS), batch, None)
        return _

    lax.fori_loop(0, EPT // ECH, step_fn, None)
    plsc.subcore_barrier()

    def wb(i, _):
        o = pl.multiple_of(base + i * 128, 8)
        pltpu.sync_copy(acc.at[pl.ds(o, 128)], zrow)
        pltpu.sync_copy(zrow, out.at[pl.ds(o, 128)])
        return _

    lax.fori_loop(0, RPT // 128, wb, None)


_conv_scratch = [
    pltpu.VMEM_SHARED((N, 16), jnp.float32),
    pltpu.VMEM_SHARED((NACC, 16), jnp.float32),
    pltpu.VMEM((125, 16), jnp.float32),
    pltpu.VMEM((ECH,), jnp.int32),
    pltpu.VMEM((ECH,), jnp.int32),
    pltpu.VMEM((NB2, LS, 16), jnp.float32),
    pltpu.VMEM((128, 16), jnp.float32),
    pltpu.SemaphoreType.DMA,
    pltpu.SemaphoreType.DMA,
]


def _spmem_pass(tab_hbm, src1, dst1, out, tabsp, acc, sbuf, idx_s, idx_d,
                rbuf, zrow, semg, sems, s):
    nrs = N // NS

    def stage(i, _):
        r0 = s * nrs + i * 125
        pltpu.sync_copy(tab_hbm.at[pl.ds(r0, 125)], sbuf)
        pltpu.sync_copy(sbuf, tabsp.at[pl.ds(r0, 125)])
        return _

    lax.fori_loop(0, nrs // 125, stage, None)
    _edge_pass(tabsp, src1, dst1, out, acc, idx_s, idx_d, rbuf, zrow,
               semg, sems, s)


def _conv1_body(tabs, src1, dst1, s1, acc, idx_s, idx_d, rbuf, zrow,
                semg, sems):
    c = lax.axis_index("c")
    s = lax.axis_index("s")
    for g in range(2):
        _edge_pass(tabs.at[g, c], src1.at[g], dst1.at[g], s1.at[g, c],
                   acc, idx_s, idx_d, rbuf, zrow, semg, sems, s)
        if g == 0:
            plsc.subcore_barrier()


_conv1_kernel = functools.partial(
    pl.kernel,
    out_type=jax.ShapeDtypeStruct((2, NC, NACC, 32), jnp.float32),
    mesh=_mesh,
    compiler_params=_sc_params,
    scratch_types=[
        pltpu.VMEM_SHARED((NACC, 32), jnp.float32),
        pltpu.VMEM((ECH,), jnp.int32),
        pltpu.VMEM((ECH,), jnp.int32),
        pltpu.VMEM((2, LS, 32), jnp.float32),
        pltpu.VMEM((128, 32), jnp.float32),
        pltpu.SemaphoreType.DMA,
        pltpu.SemaphoreType.DMA,
    ],
)(_conv1_body)


def _conv2_body(tabs, src1, dst1, s2, tabsp, acc, sbuf, idx_s, idx_d, rbuf,
                zrow, semg, sems):
    c = lax.axis_index("c")
    s = lax.axis_index("s")
    for p in range(2):
        _spmem_pass(tabs.at[c, p], src1.at[c], dst1.at[c], s2.at[c, p],
                    tabsp, acc, sbuf, idx_s, idx_d, rbuf, zrow,
                    semg, sems, s)
        if p == 0:
            plsc.subcore_barrier()


_conv2_kernel = functools.partial(
    pl.kernel,
    out_type=jax.ShapeDtypeStruct((NC, 2, NACC, 16), jnp.float32),
    mesh=_mesh,
    compiler_params=_sc_params,
    scratch_types=_conv_scratch,
)(_conv2_body)


def _pool_body(v32, bat1, pool_out, cnt_out, acc, cacc, idx, vbuf, ones_v,
               zrow, semg, sems):
    c = lax.axis_index("c")
    s = lax.axis_index("s")
    _zero_vmem(zrow, GPT * 32)
    pltpu.sync_copy(zrow, acc.at[pl.ds(s * GPT, GPT)])
    _zero_vmem(ones_v, PCH)
    pltpu.sync_copy(ones_v.at[pl.ds(0, GPT)], cacc.at[pl.ds(s * GPT, GPT)])
    one = jnp.ones((16,), jnp.float32)

    def of(i, _):
        ones_v[pl.ds(pl.multiple_of(i * 16, 16), 16)] = one
        return _

    lax.fori_loop(0, PCH // 16, of, None)
    plsc.subcore_barrier()
    pltpu.sync_copy(bat1.at[c, pl.ds(s * NPT, NPT)], idx)

    def pb(b, _):
        for j in range(2):
            bb = b * 2 + j
            o = pl.multiple_of(bb * PCH, 8)
            pltpu.async_copy(v32.at[c, pl.ds(s * NPT + o, PCH)],
                             vbuf.at[j], semg).wait()
            pltpu.async_copy(vbuf.at[j], acc.at[idx.at[pl.ds(o, PCH)]],
                             sems, add=True).wait()
            pltpu.async_copy(ones_v, cacc.at[idx.at[pl.ds(o, PCH)]],
                             sems, add=True).wait()
        return _

    lax.fori_loop(0, NPT // (2 * PCH), pb, None)
    plsc.subcore_barrier()
    pltpu.sync_copy(acc.at[pl.ds(s * GPT, GPT)], zrow)
    pltpu.sync_copy(zrow, pool_out.at[c, pl.ds(s * GPT, GPT)])
    pltpu.sync_copy(cacc.at[pl.ds(s * GPT, GPT)], ones_v.at[pl.ds(0, GPT)])
    pltpu.sync_copy(ones_v.at[pl.ds(0, GPT)],
                    cnt_out.at[c, pl.ds(s * GPT, GPT)])


_pool_kernel = functools.partial(
    pl.kernel,
    out_type=(jax.ShapeDtypeStruct((NC, GACC, 32), jnp.float32),
              jax.ShapeDtypeStruct((NC, GACC), jnp.float32)),
    mesh=_mesh,
    compiler_params=_sc_params,
    scratch_types=[
        pltpu.VMEM_SHARED((GACC, 32), jnp.float32),
        pltpu.VMEM_SHARED((GACC,), jnp.float32),
        pltpu.VMEM((NPT,), jnp.int32),
        pltpu.VMEM((2, PCH, 32), jnp.float32),
        pltpu.VMEM((PCH,), jnp.float32),
        pltpu.VMEM((GPT, 32), jnp.float32),
        pltpu.SemaphoreType.DMA,
        pltpu.SemaphoreType.DMA,
    ],
)(_pool_body)


def _mid_body(y_ref, w1_ref, b1_ref, wmu_ref, u_ref):
    y = y_ref[0]
    h = jnp.maximum(
        jnp.dot(y, w1_ref[...], preferred_element_type=jnp.float32)
        + b1_ref[...], 0.0)
    u_ref[0] = jnp.dot(h, wmu_ref[...], preferred_element_type=jnp.float32)


def _mid_call(y1pre, w1p, b1, wmu):
    blk = 2000
    return pl.pallas_call(
        _mid_body,
        grid=(2, N // blk),
        in_specs=[
            pl.BlockSpec((1, blk, DP), lambda g, i: (g, i, 0)),
            pl.BlockSpec((DP, HID), lambda g, i: (0, 0)),
            pl.BlockSpec((1, HID), lambda g, i: (0, 0)),
            pl.BlockSpec((HID, LAT), lambda g, i: (0, 0)),
        ],
        out_specs=pl.BlockSpec((1, blk, LAT), lambda g, i: (g, i, 0)),
        out_shape=jax.ShapeDtypeStruct((2, N, LAT), jnp.float32),
    )(y1pre, w1p, b1.reshape(1, HID), wmu)


def _head_body(h_ref, r1_ref, rb1_ref, r2_ref, rb2_ref, r3_ref, rb3_ref,
               o_ref):
    h = jnp.maximum(
        jnp.dot(h_ref[...], r1_ref[...], preferred_element_type=jnp.float32)
        + rb1_ref[...], 0.0)
    h = jnp.maximum(
        jnp.dot(h, r2_ref[...], preferred_element_type=jnp.float32)
        + rb2_ref[...], 0.0)
    o_ref[...] = (
        jnp.dot(h, r3_ref[...], preferred_element_type=jnp.float32)
        + rb3_ref[...])


def _head_call(hcat, R1, rb1, R2, rb2, R3, rb3):
    return pl.pallas_call(
        _head_body,
        out_shape=jax.ShapeDtypeStruct((G, 2), jnp.float32),
    )(hcat, R1, rb1.reshape(1, -1), R2, rb2.reshape(1, -1), R3,
      rb3.reshape(1, -1))


def _pad_to(a, n, lo, hi):
    pad = lo + jnp.arange(n - a.shape[0], dtype=a.dtype) % (hi - lo)
    return jnp.concatenate([a, pad])


def kernel(xA, edge_indexA, batchA, xB, edge_indexB, batchB, W1, b1, Wmu, bmu,
           Wstd, bstd, R1, rb1, R2, rb2, R3, rb3):
    src1 = jnp.stack([_pad_to(edge_indexA[0], EP, 0, N),
                      _pad_to(edge_indexB[0], EP, 0, N)])
    dst1 = jnp.stack([_pad_to(edge_indexA[1], EP, N, NACC),
                      _pad_to(edge_indexB[1], EP, N, NACC)])
    bat1 = jnp.stack([_pad_to(batchA, NPAD, G, GACC),
                      _pad_to(batchB, NPAD, G, GACC)])

    degs = _deg_kernel(dst1)[:, :N]
    dinv = lax.rsqrt(degs + 1.0)

    x = jnp.stack([xA, xB])
    xp = jnp.pad(x * dinv[:, :, None], ((0, 0), (0, 0), (0, DP - D)))
    tabs1 = jnp.stack([xp[:, :, :32], xp[:, :, 32:]], axis=1)
    s1 = _conv1_kernel(tabs1, src1, dst1)
    s1f = jnp.concatenate([s1[:, 0, :N], s1[:, 1, :N]], axis=-1)
    y1pre = dinv[:, :, None] * (s1f + xp)

    w1p = jnp.pad(W1, ((0, DP - D), (0, 0)))
    u = _mid_call(y1pre, w1p, b1, Wmu)
    up = dinv[:, :, None] * u

    uph = jnp.stack([up[:, :, :16], up[:, :, 16:]], axis=1)
    s2h = _conv2_kernel(uph, src1, dst1)
    s2 = jnp.concatenate([s2h[:, 0, :N], s2h[:, 1, :N]], axis=-1)
    v = dinv[:, :, None] * (s2 + up)
    vp = jnp.pad(v, ((0, 0), (0, NPAD - N), (0, 0)))
    outs = _pool_kernel(vp, bat1)
    pooled, cnts = (outs[0], outs[1]) if outs[0].ndim == 3 else (outs[1],
                                                                 outs[0])
    sums = pooled[:, :G, :]
    cnt = cnts[:, :G]
    z = (sums + cnt[:, :, None] * bmu) / jnp.maximum(cnt, 1.0)[:, :, None]

    hcat = jnp.concatenate([z[0], z[1]], axis=1)
    return _head_call(hcat, R1, rb1, R2, rb2, R3, rb3)


# speedup vs baseline: 1.3376x; 1.0540x over previous
import functools

import jax
import jax.numpy as jnp
from jax import lax
from jax.experimental import pallas as pl
from jax.experimental.pallas import tpu as pltpu
from jax.experimental.pallas import tpu_sc as plsc

N = 50000
E = 800000
D = 58
DP = 64
HID = 128
LAT = 32
G = 1024

NC = 2
NS = 16
EP = 819200
EPT = EP // NS
NACC = 51200
RPT = NACC // NS
GACC = 1152
GPT = GACC // NS
NPAD = 51200
NPT = NPAD // NS

LS = 256
NB2 = 4
ECH = 2048
DCH = 2048
PCH = 320

_mesh = plsc.VectorSubcoreMesh(
    core_axis_name="c", subcore_axis_name="s", num_cores=NC, num_subcores=NS)
_sc_params = pltpu.CompilerParams(use_tc_tiling_on_sc=False)


def _zero_vmem(ref, n_f32):
    z = jnp.zeros((16,), jnp.float32)
    if ref.ndim == 1:
        def zb(i, _):
            ref[pl.ds(pl.multiple_of(i * 16, 16), 16)] = z
            return _
        lax.fori_loop(0, n_f32 // 16, zb, None)
    else:
        rows, cols = ref.shape

        def zb(r, _):
            for j in range(cols // 16):
                ref[r, pl.ds(j * 16, 16)] = z
            return _
        lax.fori_loop(0, rows, zb, None)


def _deg_body(dst1, deg_out, acc, idx, ones_v, zb, sem):
    c = lax.axis_index("c")
    s = lax.axis_index("s")
    base = pl.multiple_of(s * RPT, 8)
    _zero_vmem(zb, RPT)
    one = jnp.ones((16,), jnp.float32)

    def of(i, _):
        ones_v[pl.ds(pl.multiple_of(i * 16, 16), 16)] = one
        return _

    lax.fori_loop(0, DCH // 16, of, None)
    pltpu.sync_copy(zb, acc.at[pl.ds(base, RPT)])
    plsc.subcore_barrier()
    pltpu.sync_copy(dst1.at[c, pl.ds(s * EPT, EPT)], idx)
    def db(b, _):
        pltpu.async_copy(
            ones_v,
            acc.at[idx.at[pl.ds(pl.multiple_of(b * DCH, 8), DCH)]],
            sem, add=True).wait()
        return _

    lax.fori_loop(0, EPT // DCH, db, None)
    plsc.subcore_barrier()
    pltpu.sync_copy(acc.at[pl.ds(base, RPT)], zb)
    pltpu.sync_copy(zb, deg_out.at[c, pl.ds(base, RPT)])


_deg_kernel = functools.partial(
    pl.kernel,
    out_type=jax.ShapeDtypeStruct((NC, NACC), jnp.float32),
    mesh=_mesh,
    compiler_params=_sc_params,
    scratch_types=[
        pltpu.VMEM_SHARED((NACC,), jnp.float32),
        pltpu.VMEM((EPT,), jnp.int32),
        pltpu.VMEM((DCH,), jnp.float32),
        pltpu.VMEM((RPT,), jnp.float32),
        pltpu.SemaphoreType.DMA,
    ],
)(_deg_body)


def _edge_pass(tab, src1, dst1, out, acc, idx_s, idx_d, rbuf, zrow,
               semg, sems, s):
    w = zrow.shape[1]
    nb = rbuf.shape[0]
    base = pl.multiple_of(s * RPT, 8)
    _zero_vmem(zrow, 128 * w)

    def zacc(i, _):
        pltpu.sync_copy(
            zrow, acc.at[pl.ds(pl.multiple_of(base + i * 128, 8), 128)])
        return _

    lax.fori_loop(0, RPT // 128, zacc, None)
    plsc.subcore_barrier()

    ls = rbuf.shape[1]
    gpb = nb // 2
    nbatch = ECH // (gpb * ls)

    def step_fn(step, _):
        e0 = pl.multiple_of(s * EPT + step * ECH, 8)
        pltpu.sync_copy(src1.at[pl.ds(e0, ECH)], idx_s)
        pltpu.sync_copy(dst1.at[pl.ds(e0, ECH)], idx_d)

        def fire_g(b):
            grp = (b % 2) * gpb
            r = b * gpb * ls
            return [
                pltpu.async_copy(
                    tab.at[idx_s.at[pl.ds(pl.multiple_of(r + j * ls, 8),
                                          ls)]],
                    rbuf.at[grp + j], semg)
                for j in range(gpb)
            ]

        def fire_s(b):
            grp = (b % 2) * gpb
            r = b * gpb * ls
            return [
                pltpu.async_copy(
                    rbuf.at[grp + j],
                    acc.at[idx_d.at[pl.ds(pl.multiple_of(r + j * ls, 8),
                                          ls)]],
                    sems, add=True)
                for j in range(gpb)
            ]

        gd, sd = {}, {}
        gd[0] = fire_g(0)
        for b in range(1, nbatch):
            if b >= 2:
                for d in sd.pop(b - 2):
                    d.wait()
            gd[b] = fire_g(b)
            for d in gd.pop(b - 1):
                d.wait()
            sd[b - 1] = fire_s(b - 1)
        for d in gd.pop(nbatch - 1):
            d.wait()
        sd[nbatch - 1] = fire_s(nbatch - 1)
        for b in (nbatch - 2, nbatch - 1):
            for d in sd.pop(b):
                d.wait()
        return _

    lax.fori_loop(0, EPT // ECH, step_fn, None)
    plsc.subcore_barrier()

    def wb(i, _):
        o = pl.multiple_of(base + i * 128, 8)
        pltpu.sync_copy(acc.at[pl.ds(o, 128)], zrow)
        pltpu.sync_copy(zrow, out.at[pl.ds(o, 128)])
        return _

    lax.fori_loop(0, RPT // 128, wb, None)


_conv_scratch = [
    pltpu.VMEM_SHARED((N, 16), jnp.float32),
    pltpu.VMEM_SHARED((NACC, 16), jnp.float32),
    pltpu.VMEM((125, 16), jnp.float32),
    pltpu.VMEM((ECH,), jnp.int32),
    pltpu.VMEM((ECH,), jnp.int32),
    pltpu.VMEM((NB2, LS, 16), jnp.float32),
    pltpu.VMEM((128, 16), jnp.float32),
    pltpu.SemaphoreType.DMA,
    pltpu.SemaphoreType.DMA,
]


def _spmem_pass(tab_hbm, src1, dst1, out, tabsp, acc, sbuf, idx_s, idx_d,
                rbuf, zrow, semg, sems, s):
    nrs = N // NS

    def stage(i, _):
        r0 = s * nrs + i * 125
        pltpu.sync_copy(tab_hbm.at[pl.ds(r0, 125)], sbuf)
        pltpu.sync_copy(sbuf, tabsp.at[pl.ds(r0, 125)])
        return _

    lax.fori_loop(0, nrs // 125, stage, None)
    _edge_pass(tabsp, src1, dst1, out, acc, idx_s, idx_d, rbuf, zrow,
               semg, sems, s)


def _conv1_body(tabs, src1, dst1, s1, acc, idx_s, idx_d, rbuf, zrow,
                semg, sems):
    c = lax.axis_index("c")
    s = lax.axis_index("s")
    for g in range(2):
        _edge_pass(tabs.at[g, c], src1.at[g], dst1.at[g], s1.at[g, c],
                   acc, idx_s, idx_d, rbuf, zrow, semg, sems, s)
        if g == 0:
            plsc.subcore_barrier()


_conv1_kernel = functools.partial(
    pl.kernel,
    out_type=jax.ShapeDtypeStruct((2, NC, NACC, 32), jnp.float32),
    mesh=_mesh,
    compiler_params=_sc_params,
    scratch_types=[
        pltpu.VMEM_SHARED((NACC, 32), jnp.float32),
        pltpu.VMEM((ECH,), jnp.int32),
        pltpu.VMEM((ECH,), jnp.int32),
        pltpu.VMEM((4, 128, 32), jnp.float32),
        pltpu.VMEM((128, 32), jnp.float32),
        pltpu.SemaphoreType.DMA,
        pltpu.SemaphoreType.DMA,
    ],
)(_conv1_body)


def _conv2_body(tabs, src1, dst1, s2, tabsp, acc, sbuf, idx_s, idx_d, rbuf,
                zrow, semg, sems):
    c = lax.axis_index("c")
    s = lax.axis_index("s")
    for p in range(2):
        _spmem_pass(tabs.at[c, p], src1.at[c], dst1.at[c], s2.at[c, p],
                    tabsp, acc, sbuf, idx_s, idx_d, rbuf, zrow,
                    semg, sems, s)
        if p == 0:
            plsc.subcore_barrier()


_conv2_kernel = functools.partial(
    pl.kernel,
    out_type=jax.ShapeDtypeStruct((NC, 2, NACC, 16), jnp.float32),
    mesh=_mesh,
    compiler_params=_sc_params,
    scratch_types=_conv_scratch,
)(_conv2_body)


def _pool_body(v32, bat1, pool_out, cnt_out, acc, cacc, idx, vbuf, ones_v,
               zrow, semg, sems):
    c = lax.axis_index("c")
    s = lax.axis_index("s")
    _zero_vmem(zrow, GPT * 32)
    pltpu.sync_copy(zrow, acc.at[pl.ds(s * GPT, GPT)])
    _zero_vmem(ones_v, PCH)
    pltpu.sync_copy(ones_v.at[pl.ds(0, GPT)], cacc.at[pl.ds(s * GPT, GPT)])
    one = jnp.ones((16,), jnp.float32)

    def of(i, _):
        ones_v[pl.ds(pl.multiple_of(i * 16, 16), 16)] = one
        return _

    lax.fori_loop(0, PCH // 16, of, None)
    plsc.subcore_barrier()
    pltpu.sync_copy(bat1.at[c, pl.ds(s * NPT, NPT)], idx)

    def pb(b, _):
        for j in range(2):
            bb = b * 2 + j
            o = pl.multiple_of(bb * PCH, 8)
            pltpu.async_copy(v32.at[c, pl.ds(s * NPT + o, PCH)],
                             vbuf.at[j], semg).wait()
            pltpu.async_copy(vbuf.at[j], acc.at[idx.at[pl.ds(o, PCH)]],
                             sems, add=True).wait()
            pltpu.async_copy(ones_v, cacc.at[idx.at[pl.ds(o, PCH)]],
                             sems, add=True).wait()
        return _

    lax.fori_loop(0, NPT // (2 * PCH), pb, None)
    plsc.subcore_barrier()
    pltpu.sync_copy(acc.at[pl.ds(s * GPT, GPT)], zrow)
    pltpu.sync_copy(zrow, pool_out.at[c, pl.ds(s * GPT, GPT)])
    pltpu.sync_copy(cacc.at[pl.ds(s * GPT, GPT)], ones_v.at[pl.ds(0, GPT)])
    pltpu.sync_copy(ones_v.at[pl.ds(0, GPT)],
                    cnt_out.at[c, pl.ds(s * GPT, GPT)])


_pool_kernel = functools.partial(
    pl.kernel,
    out_type=(jax.ShapeDtypeStruct((NC, GACC, 32), jnp.float32),
              jax.ShapeDtypeStruct((NC, GACC), jnp.float32)),
    mesh=_mesh,
    compiler_params=_sc_params,
    scratch_types=[
        pltpu.VMEM_SHARED((GACC, 32), jnp.float32),
        pltpu.VMEM_SHARED((GACC,), jnp.float32),
        pltpu.VMEM((NPT,), jnp.int32),
        pltpu.VMEM((2, PCH, 32), jnp.float32),
        pltpu.VMEM((PCH,), jnp.float32),
        pltpu.VMEM((GPT, 32), jnp.float32),
        pltpu.SemaphoreType.DMA,
        pltpu.SemaphoreType.DMA,
    ],
)(_pool_body)


def _mid_body(y_ref, w1_ref, b1_ref, wmu_ref, u_ref):
    y = y_ref[0]
    h = jnp.maximum(
        jnp.dot(y, w1_ref[...], preferred_element_type=jnp.float32)
        + b1_ref[...], 0.0)
    u_ref[0] = jnp.dot(h, wmu_ref[...], preferred_element_type=jnp.float32)


def _mid_call(y1pre, w1p, b1, wmu):
    blk = 2000
    return pl.pallas_call(
        _mid_body,
        grid=(2, N // blk),
        in_specs=[
            pl.BlockSpec((1, blk, DP), lambda g, i: (g, i, 0)),
            pl.BlockSpec((DP, HID), lambda g, i: (0, 0)),
            pl.BlockSpec((1, HID), lambda g, i: (0, 0)),
            pl.BlockSpec((HID, LAT), lambda g, i: (0, 0)),
        ],
        out_specs=pl.BlockSpec((1, blk, LAT), lambda g, i: (g, i, 0)),
        out_shape=jax.ShapeDtypeStruct((2, N, LAT), jnp.float32),
    )(y1pre, w1p, b1.reshape(1, HID), wmu)


def _head_body(h_ref, r1_ref, rb1_ref, r2_ref, rb2_ref, r3_ref, rb3_ref,
               o_ref):
    h = jnp.maximum(
        jnp.dot(h_ref[...], r1_ref[...], preferred_element_type=jnp.float32)
        + rb1_ref[...], 0.0)
    h = jnp.maximum(
        jnp.dot(h, r2_ref[...], preferred_element_type=jnp.float32)
        + rb2_ref[...], 0.0)
    o_ref[...] = (
        jnp.dot(h, r3_ref[...], preferred_element_type=jnp.float32)
        + rb3_ref[...])


def _head_call(hcat, R1, rb1, R2, rb2, R3, rb3):
    return pl.pallas_call(
        _head_body,
        out_shape=jax.ShapeDtypeStruct((G, 2), jnp.float32),
    )(hcat, R1, rb1.reshape(1, -1), R2, rb2.reshape(1, -1), R3,
      rb3.reshape(1, -1))


def _pad_to(a, n, lo, hi):
    pad = lo + jnp.arange(n - a.shape[0], dtype=a.dtype) % (hi - lo)
    return jnp.concatenate([a, pad])


def kernel(xA, edge_indexA, batchA, xB, edge_indexB, batchB, W1, b1, Wmu, bmu,
           Wstd, bstd, R1, rb1, R2, rb2, R3, rb3):
    src1 = jnp.stack([_pad_to(edge_indexA[0], EP, 0, N),
                      _pad_to(edge_indexB[0], EP, 0, N)])
    dst1 = jnp.stack([_pad_to(edge_indexA[1], EP, N, NACC),
                      _pad_to(edge_indexB[1], EP, N, NACC)])
    bat1 = jnp.stack([_pad_to(batchA, NPAD, G, GACC),
                      _pad_to(batchB, NPAD, G, GACC)])

    degs = _deg_kernel(dst1)[:, :N]
    dinv = lax.rsqrt(degs + 1.0)

    x = jnp.stack([xA, xB])
    xp = jnp.pad(x * dinv[:, :, None], ((0, 0), (0, 0), (0, DP - D)))
    tabs1 = jnp.stack([xp[:, :, :32], xp[:, :, 32:]], axis=1)
    s1 = _conv1_kernel(tabs1, src1, dst1)
    s1f = jnp.concatenate([s1[:, 0, :N], s1[:, 1, :N]], axis=-1)
    y1pre = dinv[:, :, None] * (s1f + xp)

    w1p = jnp.pad(W1, ((0, DP - D), (0, 0)))
    u = _mid_call(y1pre, w1p, b1, Wmu)
    up = dinv[:, :, None] * u

    uph = jnp.stack([up[:, :, :16], up[:, :, 16:]], axis=1)
    s2h = _conv2_kernel(uph, src1, dst1)
    s2 = jnp.concatenate([s2h[:, 0, :N], s2h[:, 1, :N]], axis=-1)
    v = dinv[:, :, None] * (s2 + up)
    vp = jnp.pad(v, ((0, 0), (0, NPAD - N), (0, 0)))
    outs = _pool_kernel(vp, bat1)
    pooled, cnts = (outs[0], outs[1]) if outs[0].ndim == 3 else (outs[1],
                                                                 outs[0])
    sums = pooled[:, :G, :]
    cnt = cnts[:, :G]
    z = (sums + cnt[:, :, None] * bmu) / jnp.maximum(cnt, 1.0)[:, :, None]

    hcat = jnp.concatenate([z[0], z[1]], axis=1)
    return _head_call(hcat, R1, rb1, R2, rb2, R3, rb3)
